# baseline (speedup 1.0000x reference)
import functools

import jax
import jax.numpy as jnp
from jax import lax
from jax.experimental import pallas as pl
from jax.experimental.pallas import tpu as pltpu
from jax.experimental.pallas import tpu_sc as plsc


def _make_kernel(B, L, D, NC, NS):
    NW = NC * NS
    BPW = B // NW
    CH = 40
    SPLIT = L // CH
    NCHUNK = BPW * SPLIT
    NBUF = 4
    NP = 2
    W = 4 * D
    starts = list(range(0, CH - 15, 16))
    if starts[-1] != CH - 16:
        starts.append(CH - 16)
    mesh = plsc.VectorSubcoreMesh(core_axis_name="c", subcore_axis_name="s")

    @functools.partial(
        pl.kernel,
        mesh=mesh,
        out_type=jax.ShapeDtypeStruct((B, L, D), jnp.float32),
        scratch_types=(
            [pltpu.VMEM((NCHUNK, CH), jnp.int32),
             pltpu.VMEM((NBUF, CH), jnp.int32),
             pltpu.VMEM((NBUF, CH, W), jnp.float32),
             pltpu.VMEM((NP, CH, D), jnp.float32)]
            + [pltpu.SemaphoreType.DMA] * (NBUF + NP + 1)
        ),
        compiler_params=pltpu.CompilerParams(needs_layout_passes=False),
    )
    def k(table_hbm, x_hbm, out_hbm, idx_v, widx_v, rows_v, pack_v, *sems):
        gsem = sems[:NBUF]
        osem = sems[NBUF:NBUF + NP]
        isem = sems[NBUF + NP]
        wid = lax.axis_index("s") * NC + lax.axis_index("c")
        pltpu.async_copy(x_hbm.at[wid], idx_v, isem).wait()
        lanes = lax.iota(jnp.int32, 16)

        def compute_widx(j, m):
            for st in starts:
                iv = idx_v[j, pl.ds(st, 16)]
                widx_v[m, pl.ds(st, 16)] = iv >> 2

        def fire_gather(m, b):
            pltpu.async_copy(table_hbm.at[widx_v.at[m]], rows_v.at[b],
                             gsem[b])

        def wait_gather(b):
            pltpu.make_async_copy(table_hbm.at[widx_v.at[b]], rows_v.at[b],
                                  gsem[b]).wait()

        def out_ref(j):
            return out_hbm.at[wid * BPW + j // SPLIT,
                              pl.ds((j % SPLIT) * CH, CH)]

        def fire_out(j, p):
            pltpu.async_copy(pack_v.at[p], out_ref(j), osem[p])

        def wait_out(j, p):
            pltpu.make_async_copy(pack_v.at[p], out_ref(j), osem[p]).wait()

        def extract(j, b, p):
            groups = []
            for st in starts:
                iv = idx_v[j, pl.ds(st, 16)]
                groups.append((lanes + st, (iv & 3) * D))
            zero = lanes * 0
            for kk in range(D):
                kv = zero + kk
                for row16, col0 in groups:
                    val = plsc.load_gather(rows_v.at[b], [row16, col0 + kk])
                    plsc.store_scatter(pack_v.at[p], [row16, kv], val)

        for j in range(2):
            compute_widx(j, j)
            fire_gather(j, j)

        def round_body(r, carry):
            for u in range(NBUF):
                j = r * NBUF + u
                b = u
                p = u % NP
                m2 = (u + 2) % NBUF

                @pl.when(j + 2 < NCHUNK)
                def _():
                    compute_widx(j + 2, m2)
                    fire_gather(m2, m2)

                wait_gather(b)

                @pl.when(j >= NP)
                def _():
                    wait_out(j - NP, p)

                extract(j, b, p)
                fire_out(j, p)
            return carry

        lax.fori_loop(0, NCHUNK // NBUF, round_body, 0)
        for p in range(NP):
            wait_out(NCHUNK - NP + p, p)

    return k


def kernel(table, x):
    V, D = table.shape
    B, L = x.shape
    info = plsc.get_sparse_core_info()
    NC, NS = info.num_cores, info.num_subcores
    NW = NC * NS
    assert B % NW == 0 and V % 4 == 0 and L % 40 == 0
    table_w = table.reshape(V // 4, 4 * D)
    xf = x.reshape(NW, (B // NW) * (L // 40), 40).astype(jnp.int32)
    return _make_kernel(B, L, D, NC, NS)(table_w, xf)


# speedup vs baseline: 1.0929x; 1.0929x over previous
import functools

import jax
import jax.numpy as jnp
from jax import lax
from jax.experimental import pallas as pl
from jax.experimental.pallas import tpu as pltpu
from jax.experimental.pallas import tpu_sc as plsc


def _make_kernel(B, L, D, NC, NS):
    NW = NC * NS
    BPW = B // NW
    NI = 4
    NB = 2
    W = 4 * D
    LP = 256
    starts = list(range(0, L - 15, 16))
    if starts[-1] != L - 16:
        starts.append(L - 16)
    pad_starts = [st for st in range(L, LP - 16, 16)] + [LP - 16]
    assert all(st // 128 == (st + 15) // 128 for st in starts + pad_starts)
    mesh = plsc.VectorSubcoreMesh(core_axis_name="c", subcore_axis_name="s")

    @functools.partial(
        pl.kernel,
        mesh=mesh,
        out_type=jax.ShapeDtypeStruct((B, L, D), jnp.float32),
        scratch_types=(
            [pltpu.VMEM((NI, L), jnp.int32),
             pltpu.VMEM((NB, 2, 128), jnp.int32),
             pltpu.VMEM((NB, 2, 128, W), jnp.float32),
             pltpu.VMEM((NB, L, D), jnp.float32)]
            + [pltpu.SemaphoreType.DMA] * (2 * NB + NI)
        ),
        compiler_params=pltpu.CompilerParams(needs_layout_passes=False),
    )
    def k(table_hbm, x_hbm, out_hbm, idx_v, widx_v, rows_v, pack_v, *sems):
        gsem = sems[:NB]
        osem = sems[NB:2 * NB]
        isem = sems[2 * NB:]
        wid = lax.axis_index("s") * NC + lax.axis_index("c")
        lanes = lax.iota(jnp.int32, 16)

        def fire_idx(j, s):
            pltpu.async_copy(x_hbm.at[wid, j], idx_v.at[s], isem[s])

        def wait_idx(s):
            pltpu.make_async_copy(x_hbm.at[wid, 0], idx_v.at[s],
                                  isem[s]).wait()

        def compute_widx(s, m):
            for st in starts:
                widx_v[m, st // 128, pl.ds(st % 128, 16)] = (
                    idx_v[s, pl.ds(st, 16)] >> 2)
            for st in pad_starts:
                widx_v[m, st // 128, pl.ds(st % 128, 16)] = lanes + (st - L)

        def fire_gather(m):
            for h in range(2):
                pltpu.async_copy(table_hbm.at[widx_v.at[m, h]],
                                 rows_v.at[m, h], gsem[m])

        def wait_gather(b):
            for h in range(2):
                pltpu.make_async_copy(table_hbm.at[widx_v.at[b, h]],
                                      rows_v.at[b, h], gsem[b]).wait()

        def fire_out(j, p):
            pltpu.async_copy(pack_v.at[p], out_hbm.at[wid * BPW + j], osem[p])

        def wait_out(j, p):
            pltpu.make_async_copy(pack_v.at[p], out_hbm.at[wid * BPW + j],
                                  osem[p]).wait()

        def extract(s, b):
            groups = []
            for st in starts:
                iv = idx_v[s, pl.ds(st, 16)]
                groups.append((st, lanes + st % 128, (iv & 3) * D))
            zero = lanes * 0
            for kk in range(D):
                kv = zero + kk
                for st, row16, col0 in groups:
                    val = plsc.load_gather(rows_v.at[b, st // 128],
                                           [row16, col0 + kk])
                    plsc.store_scatter(pack_v.at[b], [lanes + st, kv], val)

        for j in range(3):
            fire_idx(j, j)
        for j in range(2):
            wait_idx(j)
            compute_widx(j, j)
            fire_gather(j)

        def round_body(r, carry):
            for u in range(NI):
                j = r * NI + u
                b = u % NB

                @pl.when(j + 3 < BPW)
                def _():
                    fire_idx(j + 3, (u + 3) % NI)

                wait_gather(b)

                @pl.when(j >= NB)
                def _():
                    wait_out(j - NB, b)

                extract(u, b)

                @pl.when(j + NB < BPW)
                def _():
                    wait_idx((u + NB) % NI)
                    compute_widx((u + NB) % NI, b)
                    fire_gather(b)

                fire_out(j, b)
            return carry

        lax.fori_loop(0, BPW // NI, round_body, 0)
        for p in range(NB):
            wait_out(BPW - NB + p, p)

    return k


def kernel(table, x):
    V, D = table.shape
    B, L = x.shape
    info = plsc.get_sparse_core_info()
    NC, NS = info.num_cores, info.num_subcores
    NW = NC * NS
    assert B % NW == 0 and V % 4 == 0
    table_w = table.reshape(V // 4, 4 * D)
    xf = x.reshape(NW, B // NW, L).astype(jnp.int32)
    return _make_kernel(B, L, D, NC, NS)(table_w, xf)


# speedup vs baseline: 1.9412x; 1.7763x over previous
import functools

import jax
import jax.numpy as jnp
from jax import lax
from jax.experimental import pallas as pl
from jax.experimental.pallas import tpu as pltpu
from jax.experimental.pallas import tpu_sc as plsc


def _gather_kernel(n, D, NC, NS, C, NBUF):
    NW = NC * NS
    n_w = n // NW
    n_chunks = n_w // C
    n_rounds = n_chunks // NBUF
    mesh = plsc.VectorSubcoreMesh(core_axis_name="c", subcore_axis_name="s")

    @functools.partial(
        pl.kernel,
        mesh=mesh,
        out_type=jax.ShapeDtypeStruct((n, D), jnp.float32),
        scratch_types=(
            [pltpu.VMEM((n_chunks, C), jnp.int32),
             pltpu.VMEM((NBUF, C, D), jnp.float32)]
            + [pltpu.SemaphoreType.DMA] * (2 * NBUF)
        ),
        compiler_params=pltpu.CompilerParams(use_tc_tiling_on_sc=False),
    )
    def k(table_hbm, x_hbm, out_hbm, idx_v, rows_v, *sems):
        gsem = sems[:NBUF]
        osem = sems[NBUF:]
        wid = lax.axis_index("s") * NC + lax.axis_index("c")
        base = wid * n_w
        pltpu.sync_copy(x_hbm.at[wid], idx_v)

        for b in range(NBUF):
            pltpu.async_copy(table_hbm.at[idx_v.at[b]], rows_v.at[b], gsem[b])

        def round_body(r, carry):
            s0 = r * NBUF
            for b in range(NBUF):
                pltpu.make_async_copy(
                    table_hbm.at[idx_v.at[b]], rows_v.at[b], gsem[b]).wait()
                pltpu.async_copy(
                    rows_v.at[b],
                    out_hbm.at[pl.ds(base + (s0 + b) * C, C)],
                    osem[b])
            for b in range(NBUF):
                pltpu.make_async_copy(
                    rows_v.at[b],
                    out_hbm.at[pl.ds(base + (s0 + b) * C, C)],
                    osem[b]).wait()

                @pl.when(r < n_rounds - 1)
                def _():
                    pltpu.async_copy(
                        table_hbm.at[idx_v.at[s0 + NBUF + b]],
                        rows_v.at[b], gsem[b])
            return carry

        lax.fori_loop(0, n_rounds, round_body, 0)

    return k


def kernel(table, x):
    V, D = table.shape
    B, L = x.shape
    n = B * L
    info = plsc.get_sparse_core_info()
    NC, NS = info.num_cores, info.num_subcores
    NW = NC * NS
    C = 512
    NBUF = 5
    assert n % (NW * C * NBUF) == 0
    xf = x.reshape(NW, n // (NW * C), C).astype(jnp.int32)
    out = _gather_kernel(n, D, NC, NS, C, NBUF)(table, xf)
    return out.reshape(B, L, D)
